# Initial kernel scaffold; baseline (speedup 1.0000x reference)
import jax
import jax.numpy as jnp
from jax.experimental import pallas as pl


def kernel(query, document, W1, b1, W2, b2, W3, b3, Wg, bg):
    raise NotImplementedError("write your pallas kernel here")


# speedup vs baseline: 7.6062x; 7.6062x over previous
import functools

import jax
import jax.numpy as jnp
from jax.experimental import pallas as pl

B, Q, D, E, NBINS = 64, 16, 8192, 300, 30


def _drmm_step(q_ref, d_ref, w1_ref, b1_ref, w2_ref, b2_ref, w3_ref, b3_ref,
               wg_ref, bg_ref, out_ref):
    eps = 1e-8
    q = q_ref[0]
    d = d_ref[0]

    qss = jnp.sum(q * q, axis=1, keepdims=True)
    qn = q * (1.0 / jnp.maximum(jnp.sqrt(qss), eps))

    dss = jnp.sum(d * d, axis=1, keepdims=True)
    inv_dn = 1.0 / jnp.maximum(jnp.sqrt(dss), eps)

    inter = jax.lax.dot_general(
        qn, d, (((1,), (1,)), ((), ())),
        preferred_element_type=jnp.float32,
        precision=jax.lax.Precision.HIGHEST)
    inter = inter * inv_dn.reshape(1, D)

    y = jnp.floor((inter + 1.0) * 0.5 * NBINS)
    y = jnp.clip(y, 0.0, NBINS - 1.0)

    cols = []
    for k in range(NBINS):
        cols.append(jnp.sum(jnp.where(y == float(k), 1.0, 0.0), axis=1,
                            keepdims=True))
    h = jnp.concatenate(cols, axis=1)

    h = jnp.log1p(h)
    w1 = w1_ref[...]
    zcols = []
    for j in range(5):
        zcols.append(jnp.sum(h * w1[j:j + 1, :], axis=1, keepdims=True))
    z = jnp.tanh(jnp.concatenate(zcols, axis=1) + b1_ref[...])
    z = jnp.tanh(jnp.sum(z * w2_ref[...], axis=1, keepdims=True)
                 + b2_ref[...])
    z = jnp.tanh(z * w3_ref[...] + b3_ref[...])

    gate = jnp.sum(q * wg_ref[...], axis=1, keepdims=True) + bg_ref[...]
    gate = gate - jnp.max(gate, axis=0, keepdims=True)
    gate = jnp.exp(gate)
    gate = gate / jnp.sum(gate, axis=0, keepdims=True)

    out_ref[...] = jnp.sum(z * gate).reshape(1, 1, 1)


@jax.jit
def kernel(query, document, W1, b1, W2, b2, W3, b3, Wg, bg):
    b1r = b1.reshape(1, 5)
    b2r = b2.reshape(1, 1)
    b3r = b3.reshape(1, 1)
    bgr = bg.reshape(1, 1)

    grid = (B,)
    out = pl.pallas_call(
        _drmm_step,
        grid=grid,
        in_specs=[
            pl.BlockSpec((1, Q, E), lambda b: (b, 0, 0)),
            pl.BlockSpec((1, D, E), lambda b: (b, 0, 0)),
            pl.BlockSpec((5, NBINS), lambda b: (0, 0)),
            pl.BlockSpec((1, 5), lambda b: (0, 0)),
            pl.BlockSpec((1, 5), lambda b: (0, 0)),
            pl.BlockSpec((1, 1), lambda b: (0, 0)),
            pl.BlockSpec((1, 1), lambda b: (0, 0)),
            pl.BlockSpec((1, 1), lambda b: (0, 0)),
            pl.BlockSpec((1, E), lambda b: (0, 0)),
            pl.BlockSpec((1, 1), lambda b: (0, 0)),
        ],
        out_specs=pl.BlockSpec((1, 1, 1), lambda b: (b, 0, 0)),
        out_shape=jax.ShapeDtypeStruct((B, 1, 1), jnp.float32),
    )(query, document, W1, b1r, W2, b2r, W3, b3r, Wg, bgr)
    return out[:, 0, 0]


# speedup vs baseline: 11.0814x; 1.4569x over previous
import functools

import jax
import jax.numpy as jnp
from jax.experimental import pallas as pl

B, Q, D, E, NBINS = 64, 16, 8192, 300, 30


def _drmm_step(q_ref, d_ref, w1_ref, b1_ref, w2_ref, b2_ref, w3_ref, b3_ref,
               wg_ref, bg_ref, out_ref):
    eps = 1e-8
    q = q_ref[0]
    d = d_ref[0]

    qss = jnp.sum(q * q, axis=1, keepdims=True)
    qn = q * (1.0 / jnp.maximum(jnp.sqrt(qss), eps))

    dss = jnp.sum(d * d, axis=1, keepdims=True)
    inv_dn = 1.0 / jnp.maximum(jnp.sqrt(dss), eps)

    inter = jax.lax.dot_general(
        qn, d, (((1,), (1,)), ((), ())),
        preferred_element_type=jnp.float32,
        precision=jax.lax.Precision.DEFAULT)
    inter = inter * inv_dn.reshape(1, D)

    y = jnp.floor((inter + 1.0) * 0.5 * NBINS)
    y = jnp.clip(y, 0.0, NBINS - 1.0)

    cols = []
    for k in range(NBINS):
        cols.append(jnp.sum(jnp.where(y == float(k), 1.0, 0.0), axis=1,
                            keepdims=True))
    h = jnp.concatenate(cols, axis=1)

    h = jnp.log1p(h)
    w1 = w1_ref[...]
    zcols = []
    for j in range(5):
        zcols.append(jnp.sum(h * w1[j:j + 1, :], axis=1, keepdims=True))
    z = jnp.tanh(jnp.concatenate(zcols, axis=1) + b1_ref[...])
    z = jnp.tanh(jnp.sum(z * w2_ref[...], axis=1, keepdims=True)
                 + b2_ref[...])
    z = jnp.tanh(z * w3_ref[...] + b3_ref[...])

    gate = jnp.sum(q * wg_ref[...], axis=1, keepdims=True) + bg_ref[...]
    gate = gate - jnp.max(gate, axis=0, keepdims=True)
    gate = jnp.exp(gate)
    gate = gate / jnp.sum(gate, axis=0, keepdims=True)

    out_ref[...] = jnp.sum(z * gate).reshape(1, 1, 1)


@jax.jit
def kernel(query, document, W1, b1, W2, b2, W3, b3, Wg, bg):
    b1r = b1.reshape(1, 5)
    b2r = b2.reshape(1, 1)
    b3r = b3.reshape(1, 1)
    bgr = bg.reshape(1, 1)

    grid = (B,)
    out = pl.pallas_call(
        _drmm_step,
        grid=grid,
        in_specs=[
            pl.BlockSpec((1, Q, E), lambda b: (b, 0, 0)),
            pl.BlockSpec((1, D, E), lambda b: (b, 0, 0)),
            pl.BlockSpec((5, NBINS), lambda b: (0, 0)),
            pl.BlockSpec((1, 5), lambda b: (0, 0)),
            pl.BlockSpec((1, 5), lambda b: (0, 0)),
            pl.BlockSpec((1, 1), lambda b: (0, 0)),
            pl.BlockSpec((1, 1), lambda b: (0, 0)),
            pl.BlockSpec((1, 1), lambda b: (0, 0)),
            pl.BlockSpec((1, E), lambda b: (0, 0)),
            pl.BlockSpec((1, 1), lambda b: (0, 0)),
        ],
        out_specs=pl.BlockSpec((1, 1, 1), lambda b: (b, 0, 0)),
        out_shape=jax.ShapeDtypeStruct((B, 1, 1), jnp.float32),
    )(query, document, W1, b1r, W2, b2r, W3, b3r, Wg, bgr)
    return out[:, 0, 0]
